# Initial kernel scaffold; baseline (speedup 1.0000x reference)
import jax
import jax.numpy as jnp
from jax.experimental import pallas as pl


def kernel(x, ln1_g, ln1_b, ln2_g, ln2_b, w_qkv, b_qkv, w_o, b_o, w_gate, w1, b1, w2, b2):
    raise NotImplementedError("write your pallas kernel here")


# speedup vs baseline: 1.3875x; 1.3875x over previous
import functools

import jax
import jax.numpy as jnp
from jax import lax
from jax.experimental import pallas as pl
from jax.experimental.pallas import tpu as pltpu
from jax.experimental.pallas import tpu_sc as plsc

B, S, D = 2, 2048, 1024
H = 16
DH = D // H
E = 8
TOPK = 2
DFF = 2048
NUM_CHUNKS = 2
CAP = 640
CAP_PAD = CAP * E + 8
T = S // NUM_CHUNKS * B
PAIRS = NUM_CHUNKS * T * TOPK

SC_NC, SC_NS, SC_L = 2, 16, 16
SC_NW = SC_NC * SC_NS
SC_K = 32


def _ln(x, g, b, eps=1e-5):
    mu = jnp.mean(x, axis=-1, keepdims=True)
    r = x - mu
    var = jnp.mean(r * r, axis=-1, keepdims=True)
    return r * jax.lax.rsqrt(var + eps) * g + b


def _ln_qkv_body(x_ref, g_ref, b_ref, w_ref, bias_ref, o_ref):
    h = _ln(x_ref[0], g_ref[0], b_ref[0])
    o_ref[0] = (
        jnp.dot(h, w_ref[...], preferred_element_type=jnp.float32) + bias_ref[0]
    )


def _ln_qkv(x, ln_g, ln_b, w_qkv, b_qkv):
    RB, CB = 512, 1024
    return pl.pallas_call(
        _ln_qkv_body,
        grid=(NUM_CHUNKS, T // RB, 3 * D // CB),
        in_specs=[
            pl.BlockSpec((1, RB, D), lambda c, r, k: (c, r, 0)),
            pl.BlockSpec((1, D), lambda c, r, k: (0, 0)),
            pl.BlockSpec((1, D), lambda c, r, k: (0, 0)),
            pl.BlockSpec((D, CB), lambda c, r, k: (0, k)),
            pl.BlockSpec((1, CB), lambda c, r, k: (0, k)),
        ],
        out_specs=pl.BlockSpec((1, RB, CB), lambda c, r, k: (c, r, k)),
        out_shape=jax.ShapeDtypeStruct((NUM_CHUNKS, T, 3 * D), jnp.float32),
    )(x, ln_g.reshape(1, D), ln_b.reshape(1, D), w_qkv, b_qkv.reshape(1, 3 * D))


def _attn_body(q_ref, k_ref, v_ref, o_ref):
    q = q_ref[0, 0]
    k = k_ref[0, 0]
    v = v_ref[0, 0]
    s = lax.dot_general(q, k, (((1,), (1,)), ((), ())),
                        preferred_element_type=jnp.float32) * (1.0 / 8.0)
    m = jnp.max(s, axis=-1, keepdims=True)
    p = jnp.exp(s - m)
    p = p / jnp.sum(p, axis=-1, keepdims=True)
    o_ref[0, 0] = jnp.dot(p, v, preferred_element_type=jnp.float32)


def _attention(qkvh):
    QB = 512
    return pl.pallas_call(
        _attn_body,
        grid=(NUM_CHUNKS, H, T // QB),
        in_specs=[
            pl.BlockSpec((1, 1, QB, DH), lambda c, h, qb: (c, h, qb, 0)),
            pl.BlockSpec((1, 1, T, DH), lambda c, h, qb: (c, H + h, 0, 0)),
            pl.BlockSpec((1, 1, T, DH), lambda c, h, qb: (c, 2 * H + h, 0, 0)),
        ],
        out_specs=pl.BlockSpec((1, 1, QB, DH), lambda c, h, qb: (c, h, qb, 0)),
        out_shape=jax.ShapeDtypeStruct((NUM_CHUNKS, H, T, DH), jnp.float32),
    )(qkvh, qkvh, qkvh)


def _out_ln2_body(a_ref, x_ref, w_ref, b_ref, g2_ref, b2_ref, resid_ref, moe_ref):
    o = jnp.dot(a_ref[0], w_ref[...], preferred_element_type=jnp.float32)
    resid = o + b_ref[0] + x_ref[0]
    resid_ref[0] = resid
    moe_ref[0] = _ln(resid, g2_ref[0], b2_ref[0])


def _out_ln2(attn_o, x, w_o, b_o, ln2_g, ln2_b):
    RB = 512
    return pl.pallas_call(
        _out_ln2_body,
        grid=(NUM_CHUNKS, T // RB),
        in_specs=[
            pl.BlockSpec((1, RB, D), lambda c, r: (c, r, 0)),
            pl.BlockSpec((1, RB, D), lambda c, r: (c, r, 0)),
            pl.BlockSpec((D, D), lambda c, r: (0, 0)),
            pl.BlockSpec((1, D), lambda c, r: (0, 0)),
            pl.BlockSpec((1, D), lambda c, r: (0, 0)),
            pl.BlockSpec((1, D), lambda c, r: (0, 0)),
        ],
        out_specs=[
            pl.BlockSpec((1, RB, D), lambda c, r: (c, r, 0)),
            pl.BlockSpec((1, RB, D), lambda c, r: (c, r, 0)),
        ],
        out_shape=[
            jax.ShapeDtypeStruct((NUM_CHUNKS, T, D), jnp.float32),
            jax.ShapeDtypeStruct((NUM_CHUNKS, T, D), jnp.float32),
        ],
    )(attn_o, x, w_o, b_o.reshape(1, D), ln2_g.reshape(1, D), ln2_b.reshape(1, D))


def _route_body(moe_ref, wg_ref, idx_ref, cw_ref, cnt_ref, csum_ref):
    x = moe_ref[0]
    z = jnp.dot(x, wg_ref[...], preferred_element_type=jnp.float32)
    lanes = lax.broadcasted_iota(jnp.int32, (T, E), 1)
    m1 = jnp.max(z, axis=-1, keepdims=True)
    i1 = jnp.min(jnp.where(z >= m1, lanes, E), axis=-1, keepdims=True)
    sel1 = lanes == i1
    z2 = jnp.where(sel1, -jnp.inf, z)
    m2 = jnp.max(z2, axis=-1, keepdims=True)
    i2 = jnp.min(jnp.where(z2 >= m2, lanes, E), axis=-1, keepdims=True)
    sel2 = lanes == i2
    r = jnp.exp(m2 - m1)
    w1 = 1.0 / (1.0 + r)
    w2 = 1.0 - w1
    cnt_ref[...] = sel1.astype(jnp.float32) + sel2.astype(jnp.float32)

    def blk(j, carry):
        b = cnt_ref[pl.ds(j * 256, 256), :]
        rr = lax.broadcasted_iota(jnp.int32, (256, 256), 0)
        cc = lax.broadcasted_iota(jnp.int32, (256, 256), 1)
        tril = (rr > cc).astype(jnp.float32)
        csum_ref[pl.ds(j * 256, 256), :] = (
            jnp.dot(tril, b, preferred_element_type=jnp.float32) + carry
        )
        return carry + jnp.sum(b, axis=0, keepdims=True)

    lax.fori_loop(0, T // 256, blk, jnp.zeros((1, E), jnp.float32))
    csum = csum_ref[...]
    pos1 = jnp.sum(csum * sel1, axis=-1, keepdims=True).astype(jnp.int32)
    pos2 = jnp.sum(csum * sel2, axis=-1, keepdims=True).astype(jnp.int32)
    keep1 = pos1 < CAP
    keep2 = pos2 < CAP
    base = pl.program_id(0) * CAP_PAD
    slot1 = base + i1 * CAP + jnp.minimum(pos1, CAP - 1)
    slot2 = base + i2 * CAP + jnp.minimum(pos2, CAP - 1)
    trash1 = base + E * CAP + (pos1 & 7)
    trash2 = base + E * CAP + (pos2 & 7)
    idx_ref[0] = jnp.concatenate(
        [slot1, slot2,
         jnp.where(keep1, slot1, trash1), jnp.where(keep2, slot2, trash2)],
        axis=-1,
    )
    cw_ref[0] = jnp.concatenate(
        [jnp.where(keep1, w1, 0.0), jnp.where(keep2, w2, 0.0)], axis=-1
    )


def _route(moe_in, w_gate):
    return pl.pallas_call(
        _route_body,
        grid=(NUM_CHUNKS,),
        in_specs=[
            pl.BlockSpec((1, T, D), lambda c: (c, 0, 0)),
            pl.BlockSpec((D, E), lambda c: (0, 0)),
        ],
        out_specs=[
            pl.BlockSpec((1, T, 4), lambda c: (c, 0, 0)),
            pl.BlockSpec((1, T, 2), lambda c: (c, 0, 0)),
        ],
        out_shape=[
            jax.ShapeDtypeStruct((NUM_CHUNKS, T, 4), jnp.int32),
            jax.ShapeDtypeStruct((NUM_CHUNKS, T, 2), jnp.float32),
        ],
        scratch_shapes=[
            pltpu.VMEM((T, E), jnp.float32),
            pltpu.VMEM((T, E), jnp.float32),
        ],
    )(moe_in, w_gate)


def _sc_dispatch_body(x_hbm, dst_hbm, buf_hbm, src_v, dst_v, rows_v, sem_g, sem_s):
    wid = lax.axis_index("s") * SC_NC + lax.axis_index("c")
    per_w = PAIRS // SC_NW

    def body(b, carry):
        base = wid * per_w + b * SC_K
        l16 = lax.iota(jnp.int32, 16)
        src_v[pl.ds(0, 16)] = lax.shift_right_logical(base + l16, 1)
        src_v[pl.ds(16, 16)] = lax.shift_right_logical(base + 16 + l16, 1)
        pltpu.sync_copy(dst_hbm.at[pl.ds(base, SC_K)], dst_v)
        pltpu.async_copy(x_hbm.at[src_v], rows_v, sem_g).wait()
        pltpu.async_copy(rows_v, buf_hbm.at[dst_v], sem_s).wait()
        return carry

    lax.fori_loop(0, PAIRS // SC_NW // SC_K, body, 0)


@functools.cache
def _sc_dispatch_kernel():
    return pl.kernel(
        _sc_dispatch_body,
        out_type=jax.ShapeDtypeStruct((NUM_CHUNKS * CAP_PAD, D), jnp.float32),
        mesh=plsc.VectorSubcoreMesh(
            core_axis_name="c", subcore_axis_name="s",
            num_cores=SC_NC, num_subcores=SC_NS,
        ),
        scratch_types=[
            pltpu.VMEM((SC_K,), jnp.int32),
            pltpu.VMEM((SC_K,), jnp.int32),
            pltpu.VMEM((SC_K, D), jnp.float32),
            pltpu.SemaphoreType.DMA,
            pltpu.SemaphoreType.DMA,
        ],
    )


def _sc_dispatch(x2d, dst):
    return _sc_dispatch_kernel()(x2d, dst)


def _sc_combine_body(eo_hbm, slot_hbm, g_hbm, idx_v, rows_v, sem_g):
    wid = lax.axis_index("s") * SC_NC + lax.axis_index("c")
    per_w = PAIRS // SC_NW

    def body(b, carry):
        base = wid * per_w + b * SC_K
        pltpu.sync_copy(slot_hbm.at[pl.ds(base, SC_K)], idx_v)
        pltpu.async_copy(eo_hbm.at[idx_v], rows_v, sem_g).wait()
        pltpu.sync_copy(rows_v, g_hbm.at[pl.ds(base, SC_K)])
        return carry

    lax.fori_loop(0, PAIRS // SC_NW // SC_K, body, 0)


@functools.cache
def _sc_combine_kernel():
    return pl.kernel(
        _sc_combine_body,
        out_type=jax.ShapeDtypeStruct((PAIRS, D), jnp.float32),
        mesh=plsc.VectorSubcoreMesh(
            core_axis_name="c", subcore_axis_name="s",
            num_cores=SC_NC, num_subcores=SC_NS,
        ),
        scratch_types=[
            pltpu.VMEM((SC_K,), jnp.int32),
            pltpu.VMEM((SC_K, D), jnp.float32),
            pltpu.SemaphoreType.DMA,
        ],
    )


def _sc_combine(eo_flat, slot):
    return _sc_combine_kernel()(eo_flat, slot)


def _ffn_body(buf_ref, w1_ref, b1_ref, w2_ref, b2_ref, o_ref, h_ref):
    kb = pl.program_id(2)
    h_ref[...] = jax.nn.gelu(
        jnp.dot(buf_ref[0], w1_ref[0], preferred_element_type=jnp.float32)
        + b1_ref[0]
    )
    part = jnp.dot(h_ref[...], w2_ref[0], preferred_element_type=jnp.float32)

    @pl.when(kb == 0)
    def _():
        o_ref[0] = part + b2_ref[0]

    @pl.when(kb != 0)
    def _():
        o_ref[0] = o_ref[0] + part


def _ffn(buf3, w1, b1, w2, b2):
    FB = 1024
    return pl.pallas_call(
        _ffn_body,
        grid=(NUM_CHUNKS, E, DFF // FB),
        in_specs=[
            pl.BlockSpec((1, CAP, D), lambda c, e, k: (c, e, 0)),
            pl.BlockSpec((1, D, FB), lambda c, e, k: (e, 0, k)),
            pl.BlockSpec((1, 1, FB), lambda c, e, k: (e, 0, k)),
            pl.BlockSpec((1, FB, D), lambda c, e, k: (e, k, 0)),
            pl.BlockSpec((1, 1, D), lambda c, e, k: (e, 0, 0)),
        ],
        out_specs=pl.BlockSpec((1, CAP, D), lambda c, e, k: (c, e, 0)),
        out_shape=jax.ShapeDtypeStruct((NUM_CHUNKS, CAP_PAD, D), jnp.float32),
        scratch_shapes=[pltpu.VMEM((CAP, FB), jnp.float32)],
        compiler_params=pltpu.CompilerParams(
            dimension_semantics=("arbitrary", "arbitrary", "arbitrary"),
        ),
    )(buf3, w1, b1.reshape(E, 1, DFF), w2, b2.reshape(E, 1, D))


def _mix_body(g_ref, cw_ref, resid_ref, o_ref):
    g0 = g_ref[0, :, 0, :]
    g1 = g_ref[0, :, 1, :]
    w0 = cw_ref[0][:, 0:1]
    w1 = cw_ref[0][:, 1:2]
    o_ref[0] = resid_ref[0] + g0 * w0 + g1 * w1


def _mix(gathered, cw, resid):
    RB = 512
    return pl.pallas_call(
        _mix_body,
        grid=(NUM_CHUNKS, T // RB),
        in_specs=[
            pl.BlockSpec((1, RB, 2, D), lambda c, r: (c, r, 0, 0)),
            pl.BlockSpec((1, RB, 2), lambda c, r: (c, r, 0)),
            pl.BlockSpec((1, RB, D), lambda c, r: (c, r, 0)),
        ],
        out_specs=pl.BlockSpec((1, RB, D), lambda c, r: (c, r, 0)),
        out_shape=jax.ShapeDtypeStruct((NUM_CHUNKS, T, D), jnp.float32),
    )(gathered, cw, resid)


def kernel(x, ln1_g, ln1_b, ln2_g, ln2_b, w_qkv, b_qkv, w_o, b_o, w_gate, w1, b1, w2, b2):
    xc = x.reshape(NUM_CHUNKS, T, D)
    qkv = _ln_qkv(xc, ln1_g, ln1_b, w_qkv, b_qkv)
    qkvh = qkv.reshape(NUM_CHUNKS, T, 3 * H, DH).transpose(0, 2, 1, 3)
    attn_h = _attention(qkvh)
    attn_o = attn_h.transpose(0, 2, 1, 3).reshape(NUM_CHUNKS, T, D)
    resid, moe_in = _out_ln2(attn_o, xc, w_o, b_o, ln2_g, ln2_b)
    idx, cw = _route(moe_in, w_gate)
    slot_flat = idx[:, :, 0:2].reshape(PAIRS)
    dst_flat = idx[:, :, 2:4].reshape(PAIRS)
    buf = _sc_dispatch(moe_in.reshape(NUM_CHUNKS * T, D), dst_flat)
    eo = _ffn(buf.reshape(NUM_CHUNKS, CAP_PAD, D), w1, b1, w2, b2)
    gathered = _sc_combine(eo.reshape(NUM_CHUNKS * CAP_PAD, D), slot_flat)
    out = _mix(gathered.reshape(NUM_CHUNKS, T, TOPK, D), cw, resid)
    return out.reshape(B, S, D)
